# Initial kernel scaffold; baseline (speedup 1.0000x reference)
import jax
import jax.numpy as jnp
from jax.experimental import pallas as pl


def kernel(x, Wg, bg, W1, b1, W3, b3, W2, b2, Ws1, bs1, Ws3, bs3, Ws2, bs2):
    raise NotImplementedError("write your pallas kernel here")


# speedup vs baseline: 2.8805x; 2.8805x over previous
import functools
import jax
import jax.numpy as jnp
from jax import lax
from jax.experimental import pallas as pl
from jax.experimental.pallas import tpu as pltpu

T, D, E, K, I, SI = 2048, 1024, 64, 2, 512, 1024
BM = 128
NBLK = 96
P = NBLK * BM


def _gate_route_body(x_ref, wg_ref, bg_ref,
                     wts_ref, pos_ref, blk_e_ref, nblk_ref):
    x = x_ref[...]
    logits = lax.dot_general(x, wg_ref[...], (((1,), (1,)), ((), ())),
                             preferred_element_type=jnp.float32)
    logits = logits + bg_ref[...][None, :]
    m = jnp.max(logits, axis=1, keepdims=True)
    ex = jnp.exp(logits - m)
    scores = ex / jnp.sum(ex, axis=1, keepdims=True)

    col = lax.broadcasted_iota(jnp.int32, (T, E), 1)
    m1 = jnp.max(scores, axis=1, keepdims=True)
    a1 = jnp.min(jnp.where(scores == m1, col, E), axis=1, keepdims=True)
    sc2 = jnp.where(col == a1, -jnp.inf, scores)
    m2 = jnp.max(sc2, axis=1, keepdims=True)
    a2 = jnp.min(jnp.where(sc2 == m2, col, E), axis=1, keepdims=True)

    oh0 = (col == a1).astype(jnp.float32)
    oh1 = (col == a2).astype(jnp.float32)
    ohs = oh0 + oh1

    CH = 128
    tril = (lax.broadcasted_iota(jnp.int32, (CH, CH), 0) >=
            lax.broadcasted_iota(jnp.int32, (CH, CH), 1)).astype(jnp.float32)
    cum_chunks = []
    off_run = jnp.zeros((1, E), jnp.float32)
    for c in range(T // CH):
        blk = lax.slice(ohs, (c * CH, 0), ((c + 1) * CH, E))
        pre = lax.dot_general(tril, blk, (((1,), (0,)), ((), ())),
                              preferred_element_type=jnp.float32)
        cum_chunks.append(pre + off_run)
        off_run = off_run + pre[CH - 1:CH, :]
    cum = jnp.concatenate(cum_chunks, axis=0)
    counts = off_run
    cum_ex = cum - ohs
    rank0 = jnp.sum(cum_ex * oh0, axis=1, keepdims=True)
    rank1 = jnp.sum(cum_ex * oh1, axis=1, keepdims=True)

    nblk_e = jnp.ceil(counts / BM)
    tri = (lax.broadcasted_iota(jnp.int32, (E, E), 0) <
           lax.broadcasted_iota(jnp.int32, (E, E), 1)).astype(jnp.float32)
    blk_start = lax.dot_general(nblk_e, tri, (((1,), (0,)), ((), ())),
                                preferred_element_type=jnp.float32)
    off = blk_start * BM

    pos0 = jnp.sum(off * oh0, axis=1, keepdims=True) + rank0
    pos1 = jnp.sum(off * oh1, axis=1, keepdims=True) + rank1

    wts_ref[...] = jnp.concatenate([m1, m2], axis=1)
    pos_ref[...] = jnp.concatenate([pos0, pos1], axis=1).astype(jnp.int32)

    brow = lax.broadcasted_iota(jnp.int32, (NBLK, E), 0).astype(jnp.float32)
    blk_e_ref[...] = (jnp.sum((blk_start >= 0) & (blk_start <= brow),
                              axis=1).astype(jnp.int32) - 1)
    nblk_ref[...] = jnp.sum(nblk_e, axis=1).astype(jnp.int32)


def _gate_route(x, Wg, bg):
    return pl.pallas_call(
        _gate_route_body,
        out_shape=(
            jax.ShapeDtypeStruct((T, K), jnp.float32),
            jax.ShapeDtypeStruct((T, K), jnp.int32),
            jax.ShapeDtypeStruct((NBLK,), jnp.int32),
            jax.ShapeDtypeStruct((1,), jnp.int32),
        ),
    )(x, Wg, bg)


def _expert_body(blk_e, nblk_tot, xg_ref, w1_ref, b1_ref, w3_ref, b3_ref,
                 w2_ref, b2_ref, wts_ref, out_ref):
    b = pl.program_id(0)

    @pl.when(b < nblk_tot[0])
    def _():
        xb = xg_ref[...].astype(jnp.bfloat16)
        h1 = lax.dot_general(xb, w1_ref[0], (((1,), (1,)), ((), ())),
                             preferred_element_type=jnp.float32)
        h1 = h1 + b1_ref[0]
        h3 = lax.dot_general(xb, w3_ref[0], (((1,), (1,)), ((), ())),
                             preferred_element_type=jnp.float32)
        h3 = h3 + b3_ref[0]
        h = (h1 * jax.nn.sigmoid(h1) * h3).astype(jnp.bfloat16)
        o = lax.dot_general(h, w2_ref[0], (((1,), (1,)), ((), ())),
                            preferred_element_type=jnp.float32)
        o = o + b2_ref[0]
        w = jnp.transpose(wts_ref[0])
        out_ref[...] = o * w

    @pl.when(b >= nblk_tot[0])
    def _():
        out_ref[...] = jnp.zeros_like(out_ref)


def _expert_mlp(blk_e, nblk_tot, xg, W1, b1, W3, b3, W2, b2, wts_blk):
    grid_spec = pltpu.PrefetchScalarGridSpec(
        num_scalar_prefetch=2,
        grid=(NBLK,),
        in_specs=[
            pl.BlockSpec((BM, D), lambda b, be, nb: (b, 0)),
            pl.BlockSpec((1, I, D), lambda b, be, nb: (be[b], 0, 0)),
            pl.BlockSpec((1, 1, I), lambda b, be, nb: (be[b], 0, 0)),
            pl.BlockSpec((1, I, D), lambda b, be, nb: (be[b], 0, 0)),
            pl.BlockSpec((1, 1, I), lambda b, be, nb: (be[b], 0, 0)),
            pl.BlockSpec((1, D, I), lambda b, be, nb: (be[b], 0, 0)),
            pl.BlockSpec((1, 1, D), lambda b, be, nb: (be[b], 0, 0)),
            pl.BlockSpec((1, 1, BM), lambda b, be, nb: (b, 0, 0)),
        ],
        out_specs=pl.BlockSpec((BM, D), lambda b, be, nb: (b, 0)),
    )
    return pl.pallas_call(
        _expert_body,
        grid_spec=grid_spec,
        out_shape=jax.ShapeDtypeStruct((P, D), jnp.float32),
    )(blk_e, nblk_tot, xg, W1, b1, W3, b3, W2, b2, wts_blk)


def _shared_body(x_ref, w1_ref, b1_ref, w3_ref, b3_ref, w2_ref, b2_ref, z_ref):
    xb = x_ref[...].astype(jnp.bfloat16)
    h1 = lax.dot_general(xb, w1_ref[...], (((1,), (1,)), ((), ())),
                         preferred_element_type=jnp.float32) + b1_ref[...]
    h3 = lax.dot_general(xb, w3_ref[...], (((1,), (1,)), ((), ())),
                         preferred_element_type=jnp.float32) + b3_ref[...]
    h = (h1 * jax.nn.sigmoid(h1) * h3).astype(jnp.bfloat16)
    z_ref[...] = lax.dot_general(h, w2_ref[...], (((1,), (1,)), ((), ())),
                                 preferred_element_type=jnp.float32) + b2_ref[...]


def _shared_mlp(x, Ws1, bs1, Ws3, bs3, Ws2, bs2):
    BT = 256
    return pl.pallas_call(
        _shared_body,
        grid=(T // BT,),
        in_specs=[
            pl.BlockSpec((BT, D), lambda i: (i, 0)),
            pl.BlockSpec((SI, D), lambda i: (0, 0)),
            pl.BlockSpec((1, SI), lambda i: (0, 0)),
            pl.BlockSpec((SI, D), lambda i: (0, 0)),
            pl.BlockSpec((1, SI), lambda i: (0, 0)),
            pl.BlockSpec((D, SI), lambda i: (0, 0)),
            pl.BlockSpec((1, D), lambda i: (0, 0)),
        ],
        out_specs=pl.BlockSpec((BT, D), lambda i: (i, 0)),
        out_shape=jax.ShapeDtypeStruct((T, D), jnp.float32),
    )(x, Ws1, bs1, Ws3, bs3, Ws2, bs2)


def kernel(x, Wg, bg, W1, b1, W3, b3, W2, b2, Ws1, bs1, Ws3, bs3, Ws2, bs2):
    wts_tk, pos_tk, blk_e, nblk_tot = _gate_route(x, Wg, bg)

    posf = pos_tk.reshape(-1)
    wtsf = wts_tk.reshape(-1)
    tok = jnp.arange(T * K, dtype=jnp.int32) // K
    tok_ids = jnp.zeros((P,), jnp.int32).at[posf].set(tok)
    wts_slot = jnp.zeros((P,), jnp.float32).at[posf].set(wtsf)
    xg = jnp.take(x, tok_ids, axis=0)

    o_all = _expert_mlp(blk_e, nblk_tot, xg,
                        W1.astype(jnp.bfloat16), b1.reshape(E, 1, I),
                        W3.astype(jnp.bfloat16), b3.reshape(E, 1, I),
                        W2.astype(jnp.bfloat16), b2.reshape(E, 1, D),
                        wts_slot.reshape(NBLK, 1, BM))
    z = _shared_mlp(x, Ws1.astype(jnp.bfloat16), bs1.reshape(1, SI),
                    Ws3.astype(jnp.bfloat16), bs3.reshape(1, SI),
                    Ws2.astype(jnp.bfloat16), bs2.reshape(1, D))

    y = z + jnp.take(o_all, pos_tk[:, 0], axis=0) \
          + jnp.take(o_all, pos_tk[:, 1], axis=0)
    return y
